# Initial kernel scaffold; baseline (speedup 1.0000x reference)
import jax
import jax.numpy as jnp
from jax.experimental import pallas as pl


def kernel(edge_RWSE, batch, edge_index, e2e_edge_RWSE, e_batch, e2e_edge_index, W_enc, b_enc, W_e2e, b_e2e):
    raise NotImplementedError("write your pallas kernel here")


# speedup vs baseline: 1.1338x; 1.1338x over previous
import functools

import jax
import jax.numpy as jnp
from jax import lax
from jax.experimental import pallas as pl
from jax.experimental.pallas import tpu as pltpu
from jax.experimental.pallas import tpu_sc as plsc

N = 512
E1 = 512
E2 = 4096
PE = 16
EMB = 64

NC, NS, L = 2, 16, 16
NW = NC * NS
N1 = E1 // NW
N2 = E2 // NW

_MESH = plsc.VectorSubcoreMesh(
    core_axis_name="c", subcore_axis_name="s", num_cores=NC, num_subcores=NS)


@functools.partial(
    pl.kernel,
    mesh=_MESH,
    compiler_params=pltpu.CompilerParams(use_tc_tiling_on_sc=False),
    out_type=(
        jax.ShapeDtypeStruct((E1, PE), jnp.float32),
        jax.ShapeDtypeStruct((E2, PE), jnp.float32),
    ),
    scratch_types=[
        pltpu.VMEM((N1,), jnp.int32),
        pltpu.VMEM((N1,), jnp.int32),
        pltpu.VMEM((N1,), jnp.int32),
        pltpu.VMEM((N1,), jnp.int32),
        pltpu.VMEM((N1, PE), jnp.float32),
        pltpu.VMEM((N1, PE), jnp.float32),
        pltpu.VMEM((N1, PE), jnp.float32),
        pltpu.VMEM((N2,), jnp.int32),
        pltpu.VMEM((N2,), jnp.int32),
        pltpu.VMEM((N2,), jnp.int32),
        pltpu.VMEM((N2, PE), jnp.float32),
        pltpu.SemaphoreType.DMA,
    ],
)
def _sc_gather(r1_hbm, c1_hbm, tab1_hbm, r2_hbm, c2_hbm, tab2_hbm,
               x1_hbm, x2_hbm,
               r1_v, c1_v, fa_v, fb_v, rowsa_v, rowsb_v, avg_v,
               r2_v, c2_v, f2_v, rows2_v, sem):
    wid = lax.axis_index("s") * NC + lax.axis_index("c")
    base1 = wid * N1
    base2 = wid * N2

    pltpu.sync_copy(r1_hbm.at[pl.ds(base1, N1)], r1_v)
    pltpu.sync_copy(c1_hbm.at[pl.ds(base1, N1)], c1_v)
    pltpu.sync_copy(r2_hbm.at[pl.ds(base2, N2)], r2_v)
    pltpu.sync_copy(c2_hbm.at[pl.ds(base2, N2)], c2_v)

    r = r1_v[...]
    c = c1_v[...]
    fa_v[...] = r * N + c
    fb_v[...] = c * N + r
    for j in range(N2 // L):
        sl = pl.ds(j * L, L)
        f2_v[sl] = r2_v[sl] * N + c2_v[sl]

    cp_a = pltpu.async_copy(tab1_hbm.at[fa_v], rowsa_v, sem)
    cp_b = pltpu.async_copy(tab1_hbm.at[fb_v], rowsb_v, sem)
    cp_2 = pltpu.async_copy(tab2_hbm.at[f2_v], rows2_v, sem)
    cp_a.wait()
    cp_b.wait()

    for i in range(N1):
        avg_v[i, :] = (rowsa_v[i, :] + rowsb_v[i, :]) * 0.5
    pltpu.sync_copy(avg_v, x1_hbm.at[pl.ds(base1, N1)])

    cp_2.wait()
    pltpu.sync_copy(rows2_v, x2_hbm.at[pl.ds(base2, N2)])


def _tc_body(x1_ref, w1_ref, b1_ref, x2_ref, w2_ref, b2_ref, o1_ref, o2_ref):
    o1_ref[...] = (
        jnp.dot(x1_ref[...], w1_ref[...], preferred_element_type=jnp.float32)
        + b1_ref[...]
    )
    o2_ref[...] = (
        jnp.dot(x2_ref[...], w2_ref[...], preferred_element_type=jnp.float32)
        + b2_ref[...]
    )


_tc_encode = pl.pallas_call(
    _tc_body,
    out_shape=(
        jax.ShapeDtypeStruct((E1, EMB), jnp.float32),
        jax.ShapeDtypeStruct((E2, EMB), jnp.float32),
    ),
)


def kernel(edge_RWSE, batch, edge_index, e2e_edge_RWSE, e_batch, e2e_edge_index,
           W_enc, b_enc, W_e2e, b_e2e):
    del batch, e_batch
    x1, x2 = _sc_gather(
        edge_index[0], edge_index[1], edge_RWSE,
        e2e_edge_index[0], e2e_edge_index[1], e2e_edge_RWSE,
    )
    return _tc_encode(
        x1, W_enc, b_enc.reshape(1, EMB),
        x2, W_e2e, b_e2e.reshape(1, EMB),
    )


# speedup vs baseline: 7.6458x; 6.7436x over previous
import functools

import jax
import jax.numpy as jnp
from jax import lax
from jax.experimental import pallas as pl
from jax.experimental.pallas import tpu as pltpu
from jax.experimental.pallas import tpu_sc as plsc

N = 512
E1 = 512
E2 = 4096
PE = 16
EMB = 64

NC, NS, L = 2, 16, 16
NW = NC * NS
N1 = E1 // NW
N2 = E2 // NW

_MESH = plsc.VectorSubcoreMesh(
    core_axis_name="c", subcore_axis_name="s", num_cores=NC, num_subcores=NS)


def _koffsets():
    k = lax.iota(jnp.int32, L)
    return ((k >> 3) << 21) + ((k & 7) << 7)


@functools.partial(
    pl.kernel,
    mesh=_MESH,
    compiler_params=pltpu.CompilerParams(use_tc_tiling_on_sc=False),
    out_type=(
        jax.ShapeDtypeStruct((E1 * PE,), jnp.float32),
        jax.ShapeDtypeStruct((E2 * PE,), jnp.float32),
    ),
    scratch_types=[
        pltpu.VMEM((N1,), jnp.int32),
        pltpu.VMEM((N1,), jnp.int32),
        pltpu.VMEM((2 * N1,), jnp.int32),
        pltpu.VMEM((2 * N1 * PE,), jnp.int32),
        pltpu.VMEM((2 * N1 * PE,), jnp.float32),
        pltpu.VMEM((N1 * PE,), jnp.float32),
        pltpu.VMEM((N2,), jnp.int32),
        pltpu.VMEM((N2,), jnp.int32),
        pltpu.VMEM((N2,), jnp.int32),
        pltpu.VMEM((N2 * PE,), jnp.int32),
        pltpu.VMEM((N2 * PE,), jnp.float32),
        pltpu.SemaphoreType.DMA,
        pltpu.SemaphoreType.DMA,
    ],
)
def _sc_gather(r1_hbm, c1_hbm, tab1_hbm, r2_hbm, c2_hbm, tab2_hbm,
               x1_hbm, x2_hbm,
               r1_v, c1_v, b1_v, i1_v, g1_v, avg_v,
               r2_v, c2_v, b2_v, i2_v, g2_v, sem1, sem2):
    wid = lax.axis_index("s") * NC + lax.axis_index("c")
    koff = _koffsets()

    base1 = wid * N1
    pltpu.sync_copy(r1_hbm.at[pl.ds(base1, N1)], r1_v)
    pltpu.sync_copy(c1_hbm.at[pl.ds(base1, N1)], c1_v)
    r = r1_v[...]
    c = c1_v[...]
    f_rc = r * N + c
    f_cr = c * N + r
    b1_v[pl.ds(0, N1)] = ((f_rc >> 7) << 10) + (f_rc & 127)
    b1_v[pl.ds(N1, N1)] = ((f_cr >> 7) << 10) + (f_cr & 127)
    for g in range(2 * N1 // L):
        bv = b1_v[pl.ds(g * L, L)]
        for t in range(L):
            i1_v[pl.ds((g * L + t) * PE, PE)] = bv[t] + koff
    cp1 = pltpu.async_copy(tab1_hbm.at[i1_v], g1_v, sem1)

    base2 = wid * N2
    pltpu.sync_copy(r2_hbm.at[pl.ds(base2, N2)], r2_v)
    pltpu.sync_copy(c2_hbm.at[pl.ds(base2, N2)], c2_v)
    for j in range(N2 // L):
        sl = pl.ds(j * L, L)
        f = r2_v[sl] * N + c2_v[sl]
        b2_v[sl] = ((f >> 7) << 10) + (f & 127)
    for g in range(N2 // L):
        bv = b2_v[pl.ds(g * L, L)]
        for t in range(L):
            i2_v[pl.ds((g * L + t) * PE, PE)] = bv[t] + koff
    cp2 = pltpu.async_copy(tab2_hbm.at[i2_v], g2_v, sem2)

    cp1.wait()
    for i in range(N1):
        avg_v[pl.ds(i * PE, PE)] = (
            g1_v[pl.ds(i * PE, PE)] + g1_v[pl.ds((N1 + i) * PE, PE)]
        ) * 0.5
    pltpu.sync_copy(avg_v, x1_hbm.at[pl.ds(base1 * PE, N1 * PE)])

    cp2.wait()
    pltpu.sync_copy(g2_v, x2_hbm.at[pl.ds(base2 * PE, N2 * PE)])


def _tc_body(x1_ref, w1_ref, b1_ref, x2_ref, w2_ref, b2_ref, o1_ref, o2_ref):
    o1_ref[...] = (
        jnp.dot(x1_ref[...], w1_ref[...], preferred_element_type=jnp.float32)
        + b1_ref[...]
    )
    o2_ref[...] = (
        jnp.dot(x2_ref[...], w2_ref[...], preferred_element_type=jnp.float32)
        + b2_ref[...]
    )


_PACK = 128 // PE

_tc_encode = pl.pallas_call(
    _tc_body,
    out_shape=(
        jax.ShapeDtypeStruct((E1 // _PACK, _PACK * EMB), jnp.float32),
        jax.ShapeDtypeStruct((E2 // _PACK, _PACK * EMB), jnp.float32),
    ),
)


def _phys_view(tab):
    return (tab.T.reshape(PE // 8, 8, (N * N) // 128, 128)
            .transpose(0, 2, 1, 3).reshape(-1))


def kernel(edge_RWSE, batch, edge_index, e2e_edge_RWSE, e_batch, e2e_edge_index,
           W_enc, b_enc, W_e2e, b_e2e):
    del batch, e_batch
    x1f, x2f = _sc_gather(
        edge_index[0], edge_index[1], _phys_view(edge_RWSE),
        e2e_edge_index[0], e2e_edge_index[1], _phys_view(e2e_edge_RWSE),
    )
    eye = jnp.eye(_PACK, dtype=jnp.float32)
    o1p, o2p = _tc_encode(
        x1f.reshape(E1 // _PACK, 128), jnp.kron(eye, W_enc),
        jnp.tile(b_enc, _PACK).reshape(1, _PACK * EMB),
        x2f.reshape(E2 // _PACK, 128), jnp.kron(eye, W_e2e),
        jnp.tile(b_e2e, _PACK).reshape(1, _PACK * EMB),
    )
    return o1p.reshape(E1, EMB), o2p.reshape(E2, EMB)


# speedup vs baseline: 9.3359x; 1.2211x over previous
import functools

import jax
import jax.numpy as jnp
from jax import lax
from jax.experimental import pallas as pl
from jax.experimental.pallas import tpu as pltpu
from jax.experimental.pallas import tpu_sc as plsc

N = 512
E1 = 512
E2 = 4096
PE = 16
EMB = 64

NC, NS, L = 2, 16, 16
NW = NC * NS
N1 = E1 // NW
N2 = E2 // NW
CT1 = E1 // 128
CT2 = E2 // 128

KOFF = [((k >> 3) << 21) + ((k & 7) << 7) for k in range(PE)]

_MESH = plsc.VectorSubcoreMesh(
    core_axis_name="c", subcore_axis_name="s", num_cores=NC, num_subcores=NS)


@functools.partial(
    pl.kernel,
    mesh=_MESH,
    compiler_params=pltpu.CompilerParams(use_tc_tiling_on_sc=False),
    out_type=(
        jax.ShapeDtypeStruct((2, CT1, 8, 128), jnp.float32),
        jax.ShapeDtypeStruct((2, CT2, 1024), jnp.float32),
    ),
    scratch_types=[
        pltpu.VMEM((N1,), jnp.int32),
        pltpu.VMEM((N1,), jnp.int32),
        pltpu.VMEM((2 * N1 * PE,), jnp.int32),
        pltpu.VMEM((2 * N1 * PE,), jnp.float32),
        pltpu.VMEM((8, N1), jnp.float32),
        pltpu.VMEM((8, N1), jnp.float32),
        pltpu.VMEM((N2,), jnp.int32),
        pltpu.VMEM((N2,), jnp.int32),
        pltpu.VMEM((N2,), jnp.int32),
        pltpu.VMEM((8 * N2,), jnp.int32),
        pltpu.VMEM((8 * N2,), jnp.int32),
        pltpu.VMEM((8 * N2,), jnp.float32),
        pltpu.VMEM((8 * N2,), jnp.float32),
        pltpu.SemaphoreType.DMA,
        pltpu.SemaphoreType.DMA,
        pltpu.SemaphoreType.DMA,
    ],
)
def _sc_gather(ei1_hbm, tab1_hbm, ei2_hbm, tab2_hbm,
               x1_hbm, x2_hbm,
               r1_v, c1_v, i1_v, g1_v, avga_v, avgb_v,
               r2_v, c2_v, b2_v, i2a_v, i2b_v, g2a_v, g2b_v,
               sem1, sem2a, sem2b):
    wid = lax.axis_index("s") * NC + lax.axis_index("c")

    ct1 = wid // 8
    cl1 = (wid % 8) * N1
    pltpu.sync_copy(ei1_hbm.at[pl.ds(ct1 * 256 + cl1, N1)], r1_v)
    pltpu.sync_copy(ei1_hbm.at[pl.ds(ct1 * 256 + 128 + cl1, N1)], c1_v)
    r = r1_v[...]
    c = c1_v[...]
    f_rc = r * N + c
    f_cr = c * N + r
    brc = ((f_rc >> 7) << 10) + (f_rc & 127)
    bcr = ((f_cr >> 7) << 10) + (f_cr & 127)
    for k in range(PE):
        i1_v[pl.ds(k * N1, N1)] = brc + KOFF[k]
        i1_v[pl.ds((PE + k) * N1, N1)] = bcr + KOFF[k]
    cp1 = pltpu.async_copy(tab1_hbm.at[i1_v], g1_v, sem1)

    pltpu.sync_copy(ei2_hbm.at[pl.ds(wid * 256, N2)], r2_v)
    pltpu.sync_copy(ei2_hbm.at[pl.ds(wid * 256 + 128, N2)], c2_v)
    for g in range(N2 // L):
        sl = pl.ds(g * L, L)
        f = r2_v[sl] * N + c2_v[sl]
        b2_v[sl] = ((f >> 7) << 10) + (f & 127)
    for k8 in range(8):
        for g in range(N2 // L):
            sl = pl.ds(g * L, L)
            base = b2_v[sl]
            i2a_v[pl.ds(k8 * N2 + g * L, L)] = base + KOFF[k8]
            i2b_v[pl.ds(k8 * N2 + g * L, L)] = base + KOFF[8 + k8]
    cp2a = pltpu.async_copy(tab2_hbm.at[i2a_v], g2a_v, sem2a)
    cp2b = pltpu.async_copy(tab2_hbm.at[i2b_v], g2b_v, sem2b)

    cp1.wait()
    for k in range(PE):
        row = (g1_v[pl.ds(k * N1, N1)] + g1_v[pl.ds((PE + k) * N1, N1)]) * 0.5
        if k < 8:
            avga_v[k, :] = row
        else:
            avgb_v[k - 8, :] = row
    pltpu.sync_copy(avga_v, x1_hbm.at[0, ct1, :, pl.ds(cl1, N1)])
    pltpu.sync_copy(avgb_v, x1_hbm.at[1, ct1, :, pl.ds(cl1, N1)])

    cp2a.wait()
    pltpu.sync_copy(g2a_v, x2_hbm.at[0, wid])
    cp2b.wait()
    pltpu.sync_copy(g2b_v, x2_hbm.at[1, wid])


def _tc_body(x1_ref, w1_ref, b1_ref, x2_ref, w2_ref, b2_ref, o1_ref, o2_ref):
    dn = (((0,), (0,)), ((), ()))
    o1_ref[...] = (
        lax.dot_general(w1_ref[...], x1_ref[...], dn,
                        preferred_element_type=jnp.float32)
        + b1_ref[...]
    )
    o2_ref[...] = (
        lax.dot_general(w2_ref[...], x2_ref[...], dn,
                        preferred_element_type=jnp.float32)
        + b2_ref[...]
    )


_tc_encode = pl.pallas_call(
    _tc_body,
    out_shape=(
        jax.ShapeDtypeStruct((EMB, E1), jnp.float32),
        jax.ShapeDtypeStruct((EMB, E2), jnp.float32),
    ),
)


def _phys_view(tab):
    return (tab.T.reshape(PE // 8, 8, (N * N) // 128, 128)
            .transpose(0, 2, 1, 3).reshape(-1))


def _idx_view(ei):
    e = ei.shape[1]
    return ei.reshape(2, e // 128, 128).transpose(1, 0, 2).reshape(-1)


def _xt_view(stage, e):
    return (stage.reshape(2, e // 128, 8, 128)
            .transpose(0, 2, 1, 3).reshape(PE, e))


def kernel(edge_RWSE, batch, edge_index, e2e_edge_RWSE, e_batch, e2e_edge_index,
           W_enc, b_enc, W_e2e, b_e2e):
    del batch, e_batch
    s1, s2 = _sc_gather(
        _idx_view(edge_index), _phys_view(edge_RWSE),
        _idx_view(e2e_edge_index), _phys_view(e2e_edge_RWSE),
    )
    o1t, o2t = _tc_encode(
        _xt_view(s1, E1), W_enc, b_enc.reshape(EMB, 1),
        _xt_view(s2, E2), W_e2e, b_e2e.reshape(EMB, 1),
    )
    return o1t.T, o2t.T


# speedup vs baseline: 9.6216x; 1.0306x over previous
import functools

import jax
import jax.numpy as jnp
from jax import lax
from jax.experimental import pallas as pl
from jax.experimental.pallas import tpu as pltpu
from jax.experimental.pallas import tpu_sc as plsc

N = 512
E1 = 512
E2 = 4096
PE = 16
EMB = 64

NC, NS, L = 2, 16, 16
NW = NC * NS
N1 = E1 // NW
N2 = E2 // NW
CT1 = E1 // 128
CT2 = E2 // 128

KOFF = [((k >> 3) << 21) + ((k & 7) << 7) for k in range(PE)]

_MESH = plsc.VectorSubcoreMesh(
    core_axis_name="c", subcore_axis_name="s", num_cores=NC, num_subcores=NS)


@functools.partial(
    pl.kernel,
    mesh=_MESH,
    compiler_params=pltpu.CompilerParams(use_tc_tiling_on_sc=False),
    out_type=(
        jax.ShapeDtypeStruct((2, CT1, 8, 128), jnp.float32),
        jax.ShapeDtypeStruct((2, CT2, 1024), jnp.float32),
    ),
    scratch_types=[
        pltpu.VMEM((N1,), jnp.int32),
        pltpu.VMEM((N1,), jnp.int32),
        pltpu.VMEM((2 * N1 * PE,), jnp.int32),
        pltpu.VMEM((2 * N1 * PE,), jnp.float32),
        pltpu.VMEM((8, N1), jnp.float32),
        pltpu.VMEM((8, N1), jnp.float32),
        pltpu.VMEM((N2,), jnp.int32),
        pltpu.VMEM((N2,), jnp.int32),
        pltpu.VMEM((N2,), jnp.int32),
        pltpu.VMEM((4 * N2,), jnp.int32),
        pltpu.VMEM((4 * N2,), jnp.int32),
        pltpu.VMEM((4 * N2,), jnp.int32),
        pltpu.VMEM((4 * N2,), jnp.int32),
        pltpu.VMEM((4 * N2,), jnp.float32),
        pltpu.VMEM((4 * N2,), jnp.float32),
        pltpu.VMEM((4 * N2,), jnp.float32),
        pltpu.VMEM((4 * N2,), jnp.float32),
        pltpu.SemaphoreType.DMA,
        pltpu.SemaphoreType.DMA,
        pltpu.SemaphoreType.DMA,
        pltpu.SemaphoreType.DMA,
        pltpu.SemaphoreType.DMA,
        pltpu.SemaphoreType.DMA,
    ],
)
def _sc_gather(ei1_hbm, tab1_hbm, ei2_hbm, tab2_hbm,
               x1_hbm, x2_hbm,
               r1_v, c1_v, i1_v, g1_v, avga_v, avgb_v,
               r2_v, c2_v, b2_v, i2q0, i2q1, i2q2, i2q3,
               g2q0, g2q1, g2q2, g2q3,
               semi, sem1, s2q0, s2q1, s2q2, s2q3):
    i2_v = [i2q0, i2q1, i2q2, i2q3]
    g2_v = [g2q0, g2q1, g2q2, g2q3]
    sem2 = [s2q0, s2q1, s2q2, s2q3]
    wid = lax.axis_index("s") * NC + lax.axis_index("c")
    ct1 = wid // 8
    cl1 = (wid % 8) * N1

    ld = [
        pltpu.async_copy(ei1_hbm.at[pl.ds(ct1 * 256 + cl1, N1)], r1_v, semi),
        pltpu.async_copy(ei1_hbm.at[pl.ds(ct1 * 256 + 128 + cl1, N1)], c1_v, semi),
        pltpu.async_copy(ei2_hbm.at[pl.ds(wid * 256, N2)], r2_v, semi),
        pltpu.async_copy(ei2_hbm.at[pl.ds(wid * 256 + 128, N2)], c2_v, semi),
    ]
    ld[0].wait()
    ld[1].wait()

    r = r1_v[...]
    c = c1_v[...]
    f_rc = r * N + c
    f_cr = c * N + r
    brc = ((f_rc >> 7) << 10) + (f_rc & 127)
    bcr = ((f_cr >> 7) << 10) + (f_cr & 127)
    for k in range(PE):
        i1_v[pl.ds(k * N1, N1)] = brc + KOFF[k]
        i1_v[pl.ds((PE + k) * N1, N1)] = bcr + KOFF[k]
    cp1 = pltpu.async_copy(tab1_hbm.at[i1_v], g1_v, sem1)

    ld[2].wait()
    ld[3].wait()
    for g in range(N2 // L):
        sl = pl.ds(g * L, L)
        f = r2_v[sl] * N + c2_v[sl]
        b2_v[sl] = ((f >> 7) << 10) + (f & 127)
    cp2 = []
    for q in range(4):
        for k4 in range(4):
            for g in range(N2 // L):
                sl = pl.ds(g * L, L)
                i2_v[q][pl.ds(k4 * N2 + g * L, L)] = b2_v[sl] + KOFF[q * 4 + k4]
        cp2.append(pltpu.async_copy(tab2_hbm.at[i2_v[q]], g2_v[q], sem2[q]))

    cp1.wait()
    for k in range(PE):
        row = (g1_v[pl.ds(k * N1, N1)] + g1_v[pl.ds((PE + k) * N1, N1)]) * 0.5
        if k < 8:
            avga_v[k, :] = row
        else:
            avgb_v[k - 8, :] = row
    pltpu.sync_copy(avga_v, x1_hbm.at[0, ct1, :, pl.ds(cl1, N1)])
    pltpu.sync_copy(avgb_v, x1_hbm.at[1, ct1, :, pl.ds(cl1, N1)])

    for q in range(4):
        cp2[q].wait()
        h, off = q // 2, (q % 2) * 512
        pltpu.sync_copy(g2_v[q], x2_hbm.at[h, wid, pl.ds(off, 512)])


def _tc_body(x1_ref, w1_ref, b1_ref, x2_ref, w2_ref, b2_ref, o1_ref, o2_ref):
    dn = (((0,), (0,)), ((), ()))
    o1_ref[...] = (
        lax.dot_general(w1_ref[...], x1_ref[...], dn,
                        preferred_element_type=jnp.float32)
        + b1_ref[...]
    )
    o2_ref[...] = (
        lax.dot_general(w2_ref[...], x2_ref[...], dn,
                        preferred_element_type=jnp.float32)
        + b2_ref[...]
    )


_tc_encode = pl.pallas_call(
    _tc_body,
    out_shape=(
        jax.ShapeDtypeStruct((EMB, E1), jnp.float32),
        jax.ShapeDtypeStruct((EMB, E2), jnp.float32),
    ),
)


def _phys_view(tab):
    return (tab.T.reshape(PE // 8, 8, (N * N) // 128, 128)
            .transpose(0, 2, 1, 3).reshape(-1))


def _idx_view(ei):
    e = ei.shape[1]
    return ei.reshape(2, e // 128, 128).transpose(1, 0, 2).reshape(-1)


def _xt_view(stage, e):
    return (stage.reshape(2, e // 128, 8, 128)
            .transpose(0, 2, 1, 3).reshape(PE, e))


def kernel(edge_RWSE, batch, edge_index, e2e_edge_RWSE, e_batch, e2e_edge_index,
           W_enc, b_enc, W_e2e, b_e2e):
    del batch, e_batch
    s1, s2 = _sc_gather(
        _idx_view(edge_index), _phys_view(edge_RWSE),
        _idx_view(e2e_edge_index), _phys_view(e2e_edge_RWSE),
    )
    o1t, o2t = _tc_encode(
        _xt_view(s1, E1), W_enc, b_enc.reshape(EMB, 1),
        _xt_view(s2, E2), W_e2e, b_e2e.reshape(EMB, 1),
    )
    return o1t.T, o2t.T


# speedup vs baseline: 9.6286x; 1.0007x over previous
import functools

import jax
import jax.numpy as jnp
from jax import lax
from jax.experimental import pallas as pl
from jax.experimental.pallas import tpu as pltpu
from jax.experimental.pallas import tpu_sc as plsc

N = 512
E1 = 512
E2 = 4096
PE = 16
EMB = 64

NC, NS, L = 2, 16, 16
NW = NC * NS
N1 = E1 // NW
N2 = E2 // NW
CT1 = E1 // 128
CT2 = E2 // 128

KOFF = [((k >> 3) << 21) + ((k & 7) << 7) for k in range(PE)]

_MESH = plsc.VectorSubcoreMesh(
    core_axis_name="c", subcore_axis_name="s", num_cores=NC, num_subcores=NS)


@functools.partial(
    pl.kernel,
    mesh=_MESH,
    compiler_params=pltpu.CompilerParams(use_tc_tiling_on_sc=False),
    out_type=(
        jax.ShapeDtypeStruct((2, CT1, 8, 128), jnp.float32),
        jax.ShapeDtypeStruct((2, CT2, 1024), jnp.float32),
    ),
    scratch_types=[
        pltpu.VMEM((N1,), jnp.int32),
        pltpu.VMEM((N1,), jnp.int32),
        pltpu.VMEM((2 * N1 * PE,), jnp.int32),
        pltpu.VMEM((2 * N1 * PE,), jnp.float32),
        pltpu.VMEM((8, N1), jnp.float32),
        pltpu.VMEM((8, N1), jnp.float32),
        pltpu.VMEM((N2,), jnp.int32),
        pltpu.VMEM((N2,), jnp.int32),
        pltpu.VMEM((N2,), jnp.int32),
        pltpu.VMEM((4 * N2,), jnp.int32),
        pltpu.VMEM((4 * N2,), jnp.int32),
        pltpu.VMEM((4 * N2,), jnp.int32),
        pltpu.VMEM((4 * N2,), jnp.int32),
        pltpu.VMEM((4 * N2,), jnp.float32),
        pltpu.VMEM((4 * N2,), jnp.float32),
        pltpu.VMEM((4 * N2,), jnp.float32),
        pltpu.VMEM((4 * N2,), jnp.float32),
        pltpu.SemaphoreType.DMA,
        pltpu.SemaphoreType.DMA,
        pltpu.SemaphoreType.DMA,
        pltpu.SemaphoreType.DMA,
        pltpu.SemaphoreType.DMA,
        pltpu.SemaphoreType.DMA,
    ],
)
def _sc_gather(ei1_hbm, tab1_hbm, ei2_hbm, tab2_hbm,
               x1_hbm, x2_hbm,
               r1_v, c1_v, i1_v, g1_v, avga_v, avgb_v,
               r2_v, c2_v, b2_v, i2q0, i2q1, i2q2, i2q3,
               g2q0, g2q1, g2q2, g2q3,
               semi, sem1, s2q0, s2q1, s2q2, s2q3):
    i2_v = [i2q0, i2q1, i2q2, i2q3]
    g2_v = [g2q0, g2q1, g2q2, g2q3]
    sem2 = [s2q0, s2q1, s2q2, s2q3]
    wid = lax.axis_index("s") * NC + lax.axis_index("c")
    ct1 = wid // 8
    cl1 = (wid % 8) * N1

    ld = [
        pltpu.async_copy(ei1_hbm.at[pl.ds(ct1 * 256 + cl1, N1)], r1_v, semi),
        pltpu.async_copy(ei1_hbm.at[pl.ds(ct1 * 256 + 128 + cl1, N1)], c1_v, semi),
        pltpu.async_copy(ei2_hbm.at[pl.ds(wid * 256, N2)], r2_v, semi),
        pltpu.async_copy(ei2_hbm.at[pl.ds(wid * 256 + 128, N2)], c2_v, semi),
    ]
    ld[0].wait()
    ld[1].wait()

    r = r1_v[...]
    c = c1_v[...]
    f_rc = r * N + c
    f_cr = c * N + r
    brc = ((f_rc >> 7) << 10) + (f_rc & 127)
    bcr = ((f_cr >> 7) << 10) + (f_cr & 127)
    for k in range(PE):
        i1_v[pl.ds(k * N1, N1)] = brc + KOFF[k]
        i1_v[pl.ds((PE + k) * N1, N1)] = bcr + KOFF[k]
    cp1 = pltpu.async_copy(tab1_hbm.at[i1_v], g1_v, sem1)

    ld[2].wait()
    ld[3].wait()
    for g in range(N2 // L):
        sl = pl.ds(g * L, L)
        f = r2_v[sl] * N + c2_v[sl]
        b2_v[sl] = ((f >> 7) << 10) + (f & 127)
    cp2 = []
    for q in range(4):
        for k4 in range(4):
            for g in range(N2 // L):
                sl = pl.ds(g * L, L)
                i2_v[q][pl.ds(k4 * N2 + g * L, L)] = b2_v[sl] + KOFF[q * 4 + k4]
        cp2.append(pltpu.async_copy(tab2_hbm.at[i2_v[q]], g2_v[q], sem2[q]))

    cp1.wait()
    for k in range(PE):
        row = (g1_v[pl.ds(k * N1, N1)] + g1_v[pl.ds((PE + k) * N1, N1)]) * 0.5
        if k < 8:
            avga_v[k, :] = row
        else:
            avgb_v[k - 8, :] = row
    wb = [
        pltpu.async_copy(avga_v, x1_hbm.at[0, ct1, :, pl.ds(cl1, N1)], semi),
        pltpu.async_copy(avgb_v, x1_hbm.at[1, ct1, :, pl.ds(cl1, N1)], semi),
    ]
    for q in range(4):
        cp2[q].wait()
        h, off = q // 2, (q % 2) * 512
        wb.append(pltpu.async_copy(
            g2_v[q], x2_hbm.at[h, wid, pl.ds(off, 512)], semi))
    for w in wb:
        w.wait()


def _tc_body(x1_ref, w1_ref, b1_ref, x2_ref, w2_ref, b2_ref, o1_ref, o2_ref):
    dn = (((0,), (0,)), ((), ()))
    o1_ref[...] = (
        lax.dot_general(w1_ref[...], x1_ref[...], dn,
                        preferred_element_type=jnp.float32)
        + b1_ref[...]
    )
    o2_ref[...] = (
        lax.dot_general(w2_ref[...], x2_ref[...], dn,
                        preferred_element_type=jnp.float32)
        + b2_ref[...]
    )


_tc_encode = pl.pallas_call(
    _tc_body,
    out_shape=(
        jax.ShapeDtypeStruct((EMB, E1), jnp.float32),
        jax.ShapeDtypeStruct((EMB, E2), jnp.float32),
    ),
)


def _phys_view(tab):
    return (tab.T.reshape(PE // 8, 8, (N * N) // 128, 128)
            .transpose(0, 2, 1, 3).reshape(-1))


def _idx_view(ei):
    e = ei.shape[1]
    return ei.reshape(2, e // 128, 128).transpose(1, 0, 2).reshape(-1)


def _xt_view(stage, e):
    return (stage.reshape(2, e // 128, 8, 128)
            .transpose(0, 2, 1, 3).reshape(PE, e))


def kernel(edge_RWSE, batch, edge_index, e2e_edge_RWSE, e_batch, e2e_edge_index,
           W_enc, b_enc, W_e2e, b_e2e):
    del batch, e_batch
    s1, s2 = _sc_gather(
        _idx_view(edge_index), _phys_view(edge_RWSE),
        _idx_view(e2e_edge_index), _phys_view(e2e_edge_RWSE),
    )
    o1t, o2t = _tc_encode(
        _xt_view(s1, E1), W_enc, b_enc.reshape(EMB, 1),
        _xt_view(s2, E2), W_e2e, b_e2e.reshape(EMB, 1),
    )
    return o1t.T, o2t.T


# speedup vs baseline: 9.7950x; 1.0173x over previous
import functools

import jax
import jax.numpy as jnp
from jax import lax
from jax.experimental import pallas as pl
from jax.experimental.pallas import tpu as pltpu
from jax.experimental.pallas import tpu_sc as plsc

N = 512
E1 = 512
E2 = 4096
PE = 16
EMB = 64

NC, NS, L = 2, 16, 16
NW = NC * NS
N1 = E1 // NW
N2 = E2 // NW
CT1 = E1 // 128
CT2 = E2 // 128

KOFF = [((k >> 3) << 21) + ((k & 7) << 7) for k in range(PE)]

_MESH = plsc.VectorSubcoreMesh(
    core_axis_name="c", subcore_axis_name="s", num_cores=NC, num_subcores=NS)


@functools.partial(
    pl.kernel,
    mesh=_MESH,
    compiler_params=pltpu.CompilerParams(use_tc_tiling_on_sc=False),
    out_type=(
        jax.ShapeDtypeStruct((2, CT1, 8, 128), jnp.float32),
        jax.ShapeDtypeStruct((2, CT2, 1024), jnp.float32),
    ),
    scratch_types=[
        pltpu.VMEM((N1,), jnp.int32),
        pltpu.VMEM((N1,), jnp.int32),
        pltpu.VMEM((2 * N1 * PE,), jnp.int32),
        pltpu.VMEM((2 * N1 * PE,), jnp.float32),
        pltpu.VMEM((PE, N1), jnp.float32),
        pltpu.VMEM((N2,), jnp.int32),
        pltpu.VMEM((N2,), jnp.int32),
        pltpu.VMEM((4 * N2,), jnp.int32),
        pltpu.VMEM((4 * N2,), jnp.int32),
        pltpu.VMEM((4 * N2,), jnp.int32),
        pltpu.VMEM((4 * N2,), jnp.int32),
        pltpu.VMEM((4 * N2,), jnp.float32),
        pltpu.VMEM((4 * N2,), jnp.float32),
        pltpu.VMEM((4 * N2,), jnp.float32),
        pltpu.VMEM((4 * N2,), jnp.float32),
        pltpu.SemaphoreType.DMA,
        pltpu.SemaphoreType.DMA,
        pltpu.SemaphoreType.DMA,
        pltpu.SemaphoreType.DMA,
        pltpu.SemaphoreType.DMA,
        pltpu.SemaphoreType.DMA,
    ],
)
def _sc_gather(ei1_hbm, tab1_hbm, ei2_hbm, tab2_hbm,
               x1_hbm, x2_hbm,
               r1_v, c1_v, i1_v, g1_v, avg_v,
               r2_v, c2_v, i2q0, i2q1, i2q2, i2q3,
               g2q0, g2q1, g2q2, g2q3,
               semi, sem1, s2q0, s2q1, s2q2, s2q3):
    i2_v = [i2q0, i2q1, i2q2, i2q3]
    g2_v = [g2q0, g2q1, g2q2, g2q3]
    sem2 = [s2q0, s2q1, s2q2, s2q3]
    wid = lax.axis_index("s") * NC + lax.axis_index("c")
    ct1 = wid // 8
    cl1 = (wid % 8) * N1

    ld = [
        pltpu.async_copy(ei1_hbm.at[pl.ds(ct1 * 256 + cl1, N1)], r1_v, semi),
        pltpu.async_copy(ei1_hbm.at[pl.ds(ct1 * 256 + 128 + cl1, N1)], c1_v, semi),
        pltpu.async_copy(ei2_hbm.at[pl.ds(wid * 256, N2)], r2_v, semi),
        pltpu.async_copy(ei2_hbm.at[pl.ds(wid * 256 + 128, N2)], c2_v, semi),
    ]
    ld[0].wait()
    ld[1].wait()

    r = r1_v[...]
    c = c1_v[...]
    f_rc = r * N + c
    f_cr = c * N + r
    brc = ((f_rc >> 7) << 10) + (f_rc & 127)
    bcr = ((f_cr >> 7) << 10) + (f_cr & 127)

    def i1_body(k, carry):
        koff_k = ((k >> 3) << 21) + ((k & 7) << 7)
        i1_v[pl.ds(k * N1, N1)] = brc + koff_k
        i1_v[pl.ds((PE + k) * N1, N1)] = bcr + koff_k
        return carry

    lax.fori_loop(0, PE, i1_body, 0, unroll=False)
    cp1 = pltpu.async_copy(tab1_hbm.at[i1_v], g1_v, sem1)

    ld[2].wait()
    ld[3].wait()

    def i2_body(g, carry):
        sl = pl.ds(g * L, L)
        f = r2_v[sl] * N + c2_v[sl]
        base = ((f >> 7) << 10) + (f & 127)
        for q in range(4):
            for k4 in range(4):
                i2_v[q][pl.ds((k4 * (N2 // L) + g) * L, L)] = (
                    base + KOFF[q * 4 + k4])
        return carry

    lax.fori_loop(0, N2 // L, i2_body, 0, unroll=False)
    cp2 = [pltpu.async_copy(tab2_hbm.at[i2_v[q]], g2_v[q], sem2[q])
           for q in range(4)]

    cp1.wait()

    def avg_body(k, carry):
        row = (g1_v[pl.ds(k * N1, N1)] + g1_v[pl.ds((PE + k) * N1, N1)]) * 0.5
        avg_v[k, :] = row
        return carry

    lax.fori_loop(0, PE, avg_body, 0, unroll=False)
    wb = [
        pltpu.async_copy(avg_v.at[pl.ds(0, 8), :],
                         x1_hbm.at[0, ct1, :, pl.ds(cl1, N1)], semi),
        pltpu.async_copy(avg_v.at[pl.ds(8, 8), :],
                         x1_hbm.at[1, ct1, :, pl.ds(cl1, N1)], semi),
    ]
    for q in range(4):
        cp2[q].wait()
        h, off = q // 2, (q % 2) * 512
        wb.append(pltpu.async_copy(
            g2_v[q], x2_hbm.at[h, wid, pl.ds(off, 512)], semi))
    for w in wb:
        w.wait()


def _tc_body(x1_ref, w1_ref, b1_ref, x2_ref, w2_ref, b2_ref, o1_ref, o2_ref):
    dn = (((0,), (0,)), ((), ()))
    o1_ref[...] = (
        lax.dot_general(w1_ref[...], x1_ref[...], dn,
                        preferred_element_type=jnp.float32)
        + b1_ref[...]
    )
    o2_ref[...] = (
        lax.dot_general(w2_ref[...], x2_ref[...], dn,
                        preferred_element_type=jnp.float32)
        + b2_ref[...]
    )


_tc_encode = pl.pallas_call(
    _tc_body,
    out_shape=(
        jax.ShapeDtypeStruct((EMB, E1), jnp.float32),
        jax.ShapeDtypeStruct((EMB, E2), jnp.float32),
    ),
)


def _phys_view(tab):
    return (tab.T.reshape(PE // 8, 8, (N * N) // 128, 128)
            .transpose(0, 2, 1, 3).reshape(-1))


def _idx_view(ei):
    e = ei.shape[1]
    return ei.reshape(2, e // 128, 128).transpose(1, 0, 2).reshape(-1)


def _xt_view(stage, e):
    return (stage.reshape(2, e // 128, 8, 128)
            .transpose(0, 2, 1, 3).reshape(PE, e))


def kernel(edge_RWSE, batch, edge_index, e2e_edge_RWSE, e_batch, e2e_edge_index,
           W_enc, b_enc, W_e2e, b_e2e):
    del batch, e_batch
    s1, s2 = _sc_gather(
        _idx_view(edge_index), _phys_view(edge_RWSE),
        _idx_view(e2e_edge_index), _phys_view(e2e_edge_RWSE),
    )
    o1t, o2t = _tc_encode(
        _xt_view(s1, E1), W_enc, b_enc.reshape(EMB, 1),
        _xt_view(s2, E2), W_e2e, b_e2e.reshape(EMB, 1),
    )
    return o1t.T, o2t.T
